# Initial kernel scaffold; baseline (speedup 1.0000x reference)
import functools

import jax
import jax.numpy as jnp
from jax import lax
from jax.experimental import pallas as pl
from jax.experimental.pallas import tpu as pltpu
from jax.experimental.pallas import tpu_sc as plsc

BS = 4
SP_C = 128
DN_C = 64
N_KEY = 2048
N_Q = 4096
PTS = 32
L_SEQ = N_Q * PTS

QB = 256
NQB = N_Q // QB


def _knn_body(q_ref, k_ref, i0_ref, i1_ref, w0_ref, w1_ref):
    q = q_ref[0]
    k = k_ref[0]
    d = lax.dot_general(q, k, (((1,), (1,)), ((), ())),
                        preferred_element_type=jnp.float32)
    d = (-2.0) * d
    d = d + jnp.sum(q * q, axis=1, keepdims=True)
    d = d + jnp.sum(k * k, axis=1)[None, :]
    m1 = jnp.min(d, axis=1)
    a1 = jnp.argmin(d, axis=1).astype(jnp.int32)
    it = lax.broadcasted_iota(jnp.int32, d.shape, 1)
    d2 = jnp.where(it == a1[:, None], jnp.float32(jnp.inf), d)
    m2 = jnp.min(d2, axis=1)
    a2 = jnp.argmin(d2, axis=1).astype(jnp.int32)
    r1 = 1.0 / (m1 + 1e-8)
    r2 = 1.0 / (m2 + 1e-8)
    s = r1 + r2
    i0_ref[0, 0] = a1
    i1_ref[0, 0] = a2
    w0_ref[0, 0] = r1 / s
    w1_ref[0, 0] = r2 / s


def _knn(stk_coor_bef, stk_coor):
    out3 = lambda dt: jax.ShapeDtypeStruct((BS * NQB, 1, QB), dt)
    res = pl.pallas_call(
        _knn_body,
        grid=(BS, NQB),
        in_specs=[
            pl.BlockSpec((1, QB, 64), lambda b, qi: (b, qi, 0)),
            pl.BlockSpec((1, N_KEY, 64), lambda b, qi: (b, 0, 0)),
        ],
        out_specs=[pl.BlockSpec((1, 1, QB), lambda b, qi: (b * NQB + qi, 0, 0))] * 4,
        out_shape=[out3(jnp.int32), out3(jnp.int32), out3(jnp.float32), out3(jnp.float32)],
    )(stk_coor_bef, stk_coor)
    return tuple(r.reshape(BS, N_Q) for r in res)


NC, NS = 2, 16
NW = NC * NS
W_PER_B = NW // BS
DN_PER_W = DN_C // W_PER_B
SP_PER_W = SP_C // W_PER_B
QCH = 512
NCHK = N_Q // QCH
GRP = 16
NGRP = QCH // GRP


def _sc_gather_body(dense_hbm, sparse_hbm, i0_hbm, i1_hbm, w0_hbm, w1_hbm,
                    dn_out, spg_out,
                    tab_v, i0_v, i1_v, w0_v, w1_v, out_v, srow_v, sout_v):
    wid = lax.axis_index("s") * NC + lax.axis_index("c")
    b = wid // W_PER_B
    j = wid % W_PER_B
    pltpu.sync_copy(i0_hbm.at[b], i0_v)
    pltpu.sync_copy(i1_hbm.at[b], i1_v)
    pltpu.sync_copy(w0_hbm.at[b], w0_v)
    pltpu.sync_copy(w1_hbm.at[b], w1_v)
    lanes = lax.iota(jnp.int32, 16)
    for ci in range(DN_PER_W):
        c = j * DN_PER_W + ci
        pltpu.sync_copy(dense_hbm.at[b, c], tab_v)

        def _chunk(ch, _, c=c):
            def _group(g, _):
                q0 = ch * QCH + g * GRP
                iv0 = i0_v[pl.ds(q0, GRP)] * PTS
                iv1 = i1_v[pl.ds(q0, GRP)] * PTS
                wv0 = w0_v[pl.ds(q0, GRP)]
                wv1 = w1_v[pl.ds(q0, GRP)]
                ob = g * (GRP * PTS) + lanes * PTS
                for p in range(PTS):
                    a0 = plsc.load_gather(tab_v, [iv0 + p])
                    a1 = plsc.load_gather(tab_v, [iv1 + p])
                    plsc.store_scatter(out_v, [ob + p], wv0 * a0 + wv1 * a1)
                return 0

            lax.fori_loop(0, NGRP, _group, 0)
            pltpu.sync_copy(out_v, dn_out.at[b, c, pl.ds(ch * QCH * PTS, QCH * PTS)])
            return 0

        lax.fori_loop(0, NCHK, _chunk, 0)
    for ci in range(SP_PER_W):
        c = j * SP_PER_W + ci
        pltpu.sync_copy(sparse_hbm.at[b, c], srow_v)

        def _sgroup(g, _):
            q0 = g * GRP
            a0 = plsc.load_gather(srow_v, [i0_v[pl.ds(q0, GRP)]])
            a1 = plsc.load_gather(srow_v, [i1_v[pl.ds(q0, GRP)]])
            sout_v[pl.ds(q0, GRP)] = (w0_v[pl.ds(q0, GRP)] * a0
                                      + w1_v[pl.ds(q0, GRP)] * a1)
            return 0

        lax.fori_loop(0, N_Q // GRP, _sgroup, 0)
        pltpu.sync_copy(sout_v, spg_out.at[b, c])


def _sc_gather(dense_flat, sparse_fea, i0, i1, w0, w1):
    fn = pl.kernel(
        _sc_gather_body,
        out_type=(jax.ShapeDtypeStruct((BS, DN_C, L_SEQ), jnp.float32),
                  jax.ShapeDtypeStruct((BS, SP_C, N_Q), jnp.float32)),
        mesh=plsc.VectorSubcoreMesh(core_axis_name="c", subcore_axis_name="s"),
        scratch_types=[
            pltpu.VMEM((N_KEY * PTS,), jnp.float32),
            pltpu.VMEM((N_Q,), jnp.int32),
            pltpu.VMEM((N_Q,), jnp.int32),
            pltpu.VMEM((N_Q,), jnp.float32),
            pltpu.VMEM((N_Q,), jnp.float32),
            pltpu.VMEM((QCH * PTS,), jnp.float32),
            pltpu.VMEM((N_KEY,), jnp.float32),
            pltpu.VMEM((N_Q,), jnp.float32),
        ],
    )
    return fn(dense_flat, sparse_fea, i0, i1, w0, w1)


MB = 1024


def _mlp_body(w_ref, b_ref, x_ref, o_ref):
    x = x_ref[0]
    y = lax.dot_general(w_ref[...], x, (((1,), (0,)), ((), ())),
                        preferred_element_type=jnp.float32)
    y = (y + b_ref[...]) / jnp.sqrt(jnp.float32(1.0 + 1e-5))
    o_ref[0] = jax.nn.gelu(y)


def _mlp(sp_W, sp_b, spg):
    return pl.pallas_call(
        _mlp_body,
        grid=(BS, N_Q // MB),
        in_specs=[
            pl.BlockSpec((SP_C, SP_C), lambda b, m: (0, 0)),
            pl.BlockSpec((SP_C, 1), lambda b, m: (0, 0)),
            pl.BlockSpec((1, SP_C, MB), lambda b, m: (b, 0, m)),
        ],
        out_specs=pl.BlockSpec((1, SP_C, MB), lambda b, m: (b, 0, m)),
        out_shape=jax.ShapeDtypeStruct((BS, SP_C, N_Q), jnp.float32),
    )(sp_W, sp_b.reshape(SP_C, 1), spg)


CH = 8192
NCB = L_SEQ // CH


def _conv_body(w_ref, b_ref, x_ref, xm_ref, xp_ref, o_ref):
    kb = pl.program_id(1)
    x = x_ref[0]
    prev = xm_ref[0]
    nxt = xp_ref[0]
    cm = jnp.where(kb == 0, x[:, 0:1], prev[:, CH - 1:CH])
    cp = jnp.where(kb == NCB - 1, x[:, CH - 1:CH], nxt[:, 0:1])
    xm = jnp.concatenate([cm, x[:, :CH - 1]], axis=1)
    xp = jnp.concatenate([x[:, 1:], cp], axis=1)
    dot = lambda a, bb: lax.dot_general(a, bb, (((1,), (0,)), ((), ())),
                                        preferred_element_type=jnp.float32)
    w0 = w_ref[0:64]
    w1 = w_ref[64:128]
    w2 = w_ref[128:192]
    w3 = w_ref[192:256]
    ev = dot(w0, xm) + dot(w2, x)
    od = dot(w1, x) + dot(w3, xp)
    y = jnp.stack([ev, od], axis=-1).reshape(DN_C, 2 * CH)
    y = (y + b_ref[...]) / jnp.sqrt(jnp.float32(1.0 + 1e-5))
    o_ref[0] = jax.nn.gelu(y)


def _conv(ct_W, ct_b, dn_cf):
    w_all = jnp.transpose(ct_W[:, :, ::-1], (2, 1, 0)).reshape(4 * DN_C, DN_C)
    y = pl.pallas_call(
        _conv_body,
        grid=(BS, NCB),
        in_specs=[
            pl.BlockSpec((4 * DN_C, DN_C), lambda b, k: (0, 0)),
            pl.BlockSpec((DN_C, 1), lambda b, k: (0, 0)),
            pl.BlockSpec((1, DN_C, CH), lambda b, k: (b, 0, k)),
            pl.BlockSpec((1, DN_C, CH), lambda b, k: (b, 0, jnp.maximum(k - 1, 0))),
            pl.BlockSpec((1, DN_C, CH), lambda b, k: (b, 0, jnp.minimum(k + 1, NCB - 1))),
        ],
        out_specs=pl.BlockSpec((1, DN_C, 2 * CH), lambda b, k: (b, 0, k)),
        out_shape=jax.ShapeDtypeStruct((BS, DN_C, 2 * L_SEQ), jnp.float32),
    )(w_all, ct_b.reshape(DN_C, 1), dn_cf)
    return y.reshape(BS, DN_C, N_Q, 2 * L_SEQ // N_Q)


def kernel(sparse_fea, dense_fea, stk_coor, stk_coor_bef, sp_W, sp_b, ct_W, ct_b):
    i0, i1, w0, w1 = _knn(stk_coor_bef, stk_coor)
    dense_flat = dense_fea.reshape(BS, DN_C, N_KEY * PTS)
    dn_cf, spg = _sc_gather(dense_flat, sparse_fea, i0, i1, w0, w1)
    sp = _mlp(sp_W, sp_b, spg)
    y = _conv(ct_W, ct_b, dn_cf)
    return (sp, y)


# speedup vs baseline: 3.2069x; 3.2069x over previous
import functools

import jax
import jax.numpy as jnp
from jax import lax
from jax.experimental import pallas as pl
from jax.experimental.pallas import tpu as pltpu
from jax.experimental.pallas import tpu_sc as plsc

BS = 4
SP_C = 128
DN_C = 64
N_KEY = 2048
N_Q = 4096
PTS = 32
L_SEQ = N_Q * PTS

QB = 256
NQB = N_Q // QB


def _knn_body(q_ref, k_ref, i0_ref, i1_ref, w0_ref, w1_ref):
    q = q_ref[0]
    k = k_ref[0]
    d = lax.dot_general(q, k, (((1,), (1,)), ((), ())),
                        preferred_element_type=jnp.float32)
    d = (-2.0) * d
    d = d + jnp.sum(q * q, axis=1, keepdims=True)
    d = d + jnp.sum(k * k, axis=1)[None, :]
    m1 = jnp.min(d, axis=1)
    a1 = jnp.argmin(d, axis=1).astype(jnp.int32)
    it = lax.broadcasted_iota(jnp.int32, d.shape, 1)
    d2 = jnp.where(it == a1[:, None], jnp.float32(jnp.inf), d)
    m2 = jnp.min(d2, axis=1)
    a2 = jnp.argmin(d2, axis=1).astype(jnp.int32)
    r1 = 1.0 / (m1 + 1e-8)
    r2 = 1.0 / (m2 + 1e-8)
    s = r1 + r2
    i0_ref[0, 0] = a1
    i1_ref[0, 0] = a2
    w0_ref[0, 0] = r1 / s
    w1_ref[0, 0] = r2 / s


def _knn(stk_coor_bef, stk_coor):
    out3 = lambda dt: jax.ShapeDtypeStruct((BS * NQB, 1, QB), dt)
    res = pl.pallas_call(
        _knn_body,
        grid=(BS, NQB),
        in_specs=[
            pl.BlockSpec((1, QB, 64), lambda b, qi: (b, qi, 0)),
            pl.BlockSpec((1, N_KEY, 64), lambda b, qi: (b, 0, 0)),
        ],
        out_specs=[pl.BlockSpec((1, 1, QB), lambda b, qi: (b * NQB + qi, 0, 0))] * 4,
        out_shape=[out3(jnp.int32), out3(jnp.int32), out3(jnp.float32), out3(jnp.float32)],
    )(stk_coor_bef, stk_coor)
    return tuple(r.reshape(BS, N_Q) for r in res)


NC, NS = 2, 16
NW = NC * NS
W_PER_B = NW // BS
DN_PER_W = DN_C // W_PER_B
SP_PER_W = SP_C // W_PER_B
QCH = 512
NCHK = N_Q // QCH
GRP = 16
NGRP = QCH // GRP


def _sc_gather_body(dense_hbm, sparse_hbm, i0_hbm, i1_hbm, w0_hbm, w1_hbm,
                    dn_out, spg_out,
                    tab_v, i0_v, i1_v, w0_v, w1_v, out_v, srow_v, sout_v):
    wid = lax.axis_index("s") * NC + lax.axis_index("c")
    b = wid // W_PER_B
    j = wid % W_PER_B
    pltpu.sync_copy(i0_hbm.at[b], i0_v)
    pltpu.sync_copy(i1_hbm.at[b], i1_v)
    pltpu.sync_copy(w0_hbm.at[b], w0_v)
    pltpu.sync_copy(w1_hbm.at[b], w1_v)
    lanes = lax.iota(jnp.int32, 16)
    for ci in range(DN_PER_W):
        c = j * DN_PER_W + ci
        pltpu.sync_copy(dense_hbm.at[b, c], tab_v)

        def _chunk(ch, _, c=c):
            def _group(g, _):
                q0 = ch * QCH + g * GRP
                iv0 = i0_v[pl.ds(q0, GRP)] * PTS
                iv1 = i1_v[pl.ds(q0, GRP)] * PTS
                wv0 = w0_v[pl.ds(q0, GRP)]
                wv1 = w1_v[pl.ds(q0, GRP)]
                ob = g * (GRP * PTS) + lanes * PTS
                for p in range(PTS):
                    a0 = plsc.load_gather(tab_v, [iv0 + p])
                    a1 = plsc.load_gather(tab_v, [iv1 + p])
                    plsc.store_scatter(out_v, [ob + p], wv0 * a0 + wv1 * a1)
                return 0

            lax.fori_loop(0, NGRP, _group, 0)
            pltpu.sync_copy(out_v, dn_out.at[b, c, pl.ds(ch * QCH * PTS, QCH * PTS)])
            return 0

        lax.fori_loop(0, NCHK, _chunk, 0)
    for ci in range(SP_PER_W):
        c = j * SP_PER_W + ci
        pltpu.sync_copy(sparse_hbm.at[b, c], srow_v)

        def _sgroup(g, _):
            q0 = g * GRP
            a0 = plsc.load_gather(srow_v, [i0_v[pl.ds(q0, GRP)]])
            a1 = plsc.load_gather(srow_v, [i1_v[pl.ds(q0, GRP)]])
            sout_v[pl.ds(q0, GRP)] = (w0_v[pl.ds(q0, GRP)] * a0
                                      + w1_v[pl.ds(q0, GRP)] * a1)
            return 0

        lax.fori_loop(0, N_Q // GRP, _sgroup, 0)
        pltpu.sync_copy(sout_v, spg_out.at[b, c])


def _sc_gather(dense_flat, sparse_fea, i0, i1, w0, w1):
    fn = pl.kernel(
        _sc_gather_body,
        out_type=(jax.ShapeDtypeStruct((BS, DN_C, L_SEQ), jnp.float32),
                  jax.ShapeDtypeStruct((BS, SP_C, N_Q), jnp.float32)),
        mesh=plsc.VectorSubcoreMesh(core_axis_name="c", subcore_axis_name="s"),
        compiler_params=pltpu.CompilerParams(needs_layout_passes=False),
        scratch_types=[
            pltpu.VMEM((N_KEY * PTS,), jnp.float32),
            pltpu.VMEM((N_Q,), jnp.int32),
            pltpu.VMEM((N_Q,), jnp.int32),
            pltpu.VMEM((N_Q,), jnp.float32),
            pltpu.VMEM((N_Q,), jnp.float32),
            pltpu.VMEM((QCH * PTS,), jnp.float32),
            pltpu.VMEM((N_KEY,), jnp.float32),
            pltpu.VMEM((N_Q,), jnp.float32),
        ],
    )
    return fn(dense_flat, sparse_fea, i0, i1, w0, w1)


MB = 1024


def _mlp_body(w_ref, b_ref, x_ref, o_ref):
    x = x_ref[0]
    y = lax.dot_general(w_ref[...], x, (((1,), (0,)), ((), ())),
                        preferred_element_type=jnp.float32)
    y = (y + b_ref[...]) / jnp.sqrt(jnp.float32(1.0 + 1e-5))
    o_ref[0] = jax.nn.gelu(y)


def _mlp(sp_W, sp_b, spg):
    return pl.pallas_call(
        _mlp_body,
        grid=(BS, N_Q // MB),
        in_specs=[
            pl.BlockSpec((SP_C, SP_C), lambda b, m: (0, 0)),
            pl.BlockSpec((SP_C, 1), lambda b, m: (0, 0)),
            pl.BlockSpec((1, SP_C, MB), lambda b, m: (b, 0, m)),
        ],
        out_specs=pl.BlockSpec((1, SP_C, MB), lambda b, m: (b, 0, m)),
        out_shape=jax.ShapeDtypeStruct((BS, SP_C, N_Q), jnp.float32),
    )(sp_W, sp_b.reshape(SP_C, 1), spg)


CH = 8192
NCB = L_SEQ // CH


SEG = 128


def _conv_body(w_ref, b_ref, p_ref, x_ref, xm_ref, xp_ref, o_ref):
    kb = pl.program_id(1)
    x = x_ref[0]
    prev = xm_ref[0]
    nxt = xp_ref[0]
    cm = jnp.where(kb == 0, x[:, 0:1], prev[:, CH - 1:CH])
    cp = jnp.where(kb == NCB - 1, x[:, CH - 1:CH], nxt[:, 0:1])
    xm = jnp.concatenate([cm, x[:, :CH - 1]], axis=1)
    xp = jnp.concatenate([x[:, 1:], cp], axis=1)
    dot = lambda a, bb: lax.dot_general(a, bb, (((1,), (0,)), ((), ())),
                                        preferred_element_type=jnp.float32)
    w0 = w_ref[0:64]
    w1 = w_ref[64:128]
    w2 = w_ref[128:192]
    w3 = w_ref[192:256]
    ev = dot(w0, xm) + dot(w2, x)
    od = dot(w1, x) + dot(w3, xp)
    scale = 1.0 / jnp.sqrt(jnp.float32(1.0 + 1e-5))
    bias = b_ref[...]
    ev = jax.nn.gelu((ev + bias) * scale).astype(jnp.bfloat16)
    od = jax.nn.gelu((od + bias) * scale).astype(jnp.bfloat16)
    perm = p_ref[...]
    pieces = []
    for u in range(CH // SEG):
        eo = jnp.concatenate(
            [ev[:, u * SEG:(u + 1) * SEG], od[:, u * SEG:(u + 1) * SEG]], axis=1)
        pieces.append(lax.dot_general(eo, perm, (((1,), (0,)), ((), ())),
                                      preferred_element_type=jnp.float32))
    o_ref[0] = jnp.concatenate(pieces, axis=1)


def _conv(ct_W, ct_b, dn_cf):
    w_all = jnp.transpose(ct_W[:, :, ::-1], (2, 1, 0)).reshape(4 * DN_C, DN_C)
    tt = jnp.arange(2 * SEG)
    src = tt // 2 + SEG * (tt % 2)
    perm = (jnp.arange(2 * SEG)[:, None] == src[None, :]).astype(jnp.bfloat16)
    y = pl.pallas_call(
        _conv_body,
        grid=(BS, NCB),
        in_specs=[
            pl.BlockSpec((4 * DN_C, DN_C), lambda b, k: (0, 0)),
            pl.BlockSpec((DN_C, 1), lambda b, k: (0, 0)),
            pl.BlockSpec((2 * SEG, 2 * SEG), lambda b, k: (0, 0)),
            pl.BlockSpec((1, DN_C, CH), lambda b, k: (b, 0, k)),
            pl.BlockSpec((1, DN_C, CH), lambda b, k: (b, 0, jnp.maximum(k - 1, 0))),
            pl.BlockSpec((1, DN_C, CH), lambda b, k: (b, 0, jnp.minimum(k + 1, NCB - 1))),
        ],
        out_specs=pl.BlockSpec((1, DN_C, 2 * CH), lambda b, k: (b, 0, k)),
        out_shape=jax.ShapeDtypeStruct((BS, DN_C, 2 * L_SEQ), jnp.float32),
    )(w_all, ct_b.reshape(DN_C, 1), perm, dn_cf, dn_cf, dn_cf)
    return y.reshape(BS, DN_C, N_Q, 2 * L_SEQ // N_Q)


def kernel(sparse_fea, dense_fea, stk_coor, stk_coor_bef, sp_W, sp_b, ct_W, ct_b):
    i0, i1, w0, w1 = _knn(stk_coor_bef, stk_coor)
    dense_flat = dense_fea.reshape(BS, DN_C, N_KEY * PTS)
    dn_cf, spg = _sc_gather(dense_flat, sparse_fea, i0, i1, w0, w1)
    sp = _mlp(sp_W, sp_b, spg)
    y = _conv(ct_W, ct_b, dn_cf)
    return (sp, y)


# speedup vs baseline: 4.1631x; 1.2982x over previous
import functools

import jax
import jax.numpy as jnp
from jax import lax
from jax.experimental import pallas as pl
from jax.experimental.pallas import tpu as pltpu
from jax.experimental.pallas import tpu_sc as plsc

BS = 4
SP_C = 128
DN_C = 64
N_KEY = 2048
N_Q = 4096
PTS = 32
L_SEQ = N_Q * PTS

QB = 256
NQB = N_Q // QB


def _knn_body(q_ref, k_ref, i0_ref, i1_ref, w0_ref, w1_ref):
    q = q_ref[0]
    k = k_ref[0]
    d = lax.dot_general(q, k, (((1,), (1,)), ((), ())),
                        preferred_element_type=jnp.float32)
    d = (-2.0) * d
    d = d + jnp.sum(q * q, axis=1, keepdims=True)
    d = d + jnp.sum(k * k, axis=1)[None, :]
    m1 = jnp.min(d, axis=1)
    a1 = jnp.argmin(d, axis=1).astype(jnp.int32)
    it = lax.broadcasted_iota(jnp.int32, d.shape, 1)
    d2 = jnp.where(it == a1[:, None], jnp.float32(jnp.inf), d)
    m2 = jnp.min(d2, axis=1)
    a2 = jnp.argmin(d2, axis=1).astype(jnp.int32)
    r1 = 1.0 / (m1 + 1e-8)
    r2 = 1.0 / (m2 + 1e-8)
    s = r1 + r2
    i0_ref[0, 0] = a1
    i1_ref[0, 0] = a2
    w0_ref[0, 0] = r1 / s
    w1_ref[0, 0] = r2 / s


def _knn(stk_coor_bef, stk_coor):
    out3 = lambda dt: jax.ShapeDtypeStruct((BS * NQB, 1, QB), dt)
    res = pl.pallas_call(
        _knn_body,
        grid=(BS, NQB),
        in_specs=[
            pl.BlockSpec((1, QB, 64), lambda b, qi: (b, qi, 0)),
            pl.BlockSpec((1, N_KEY, 64), lambda b, qi: (b, 0, 0)),
        ],
        out_specs=[pl.BlockSpec((1, 1, QB), lambda b, qi: (b * NQB + qi, 0, 0))] * 4,
        out_shape=[out3(jnp.int32), out3(jnp.int32), out3(jnp.float32), out3(jnp.float32)],
    )(stk_coor_bef, stk_coor)
    return tuple(r.reshape(BS, N_Q) for r in res)


NC, NS = 2, 16
NW = NC * NS
W_PER_B = NW // BS
DN_PER_W = DN_C // W_PER_B
SP_PER_W = SP_C // W_PER_B
QCH = 512
NCHK = N_Q // QCH
GRP = 16
NGRP = QCH // GRP


def _sc_gather_body(dense_hbm, sparse_hbm, i0_hbm, i1_hbm, w0_hbm, w1_hbm,
                    dn_out, spg_out,
                    tab_v, i0_v, i1_v, w0_v, w1_v, out_v, srow_v, sout_v,
                    osem):
    wid = lax.axis_index("s") * NC + lax.axis_index("c")
    b = wid // W_PER_B
    j = wid % W_PER_B
    pltpu.sync_copy(i0_hbm.at[b], i0_v)
    pltpu.sync_copy(i1_hbm.at[b], i1_v)
    pltpu.sync_copy(w0_hbm.at[b], w0_v)
    pltpu.sync_copy(w1_hbm.at[b], w1_v)
    lanes = lax.iota(jnp.int32, 16)
    csz = QCH * PTS

    def _drain_out(c):
        pltpu.make_async_copy(
            out_v.at[pl.ds(0, csz)], dn_out.at[b, c, pl.ds(0, csz)], osem
        ).wait()

    for ci in range(DN_PER_W):
        c = j * DN_PER_W + ci
        pltpu.sync_copy(dense_hbm.at[b, c], tab_v)

        def _chunk(ch, _, c=c):
            par = (ch % 2) * csz

            @pl.when(ch >= 2)
            def _():
                _drain_out(c)

            @plsc.parallel_loop(0, NGRP)
            def _group(g):
                q0 = ch * QCH + g * GRP
                iv0 = i0_v[pl.ds(q0, GRP)] * PTS
                iv1 = i1_v[pl.ds(q0, GRP)] * PTS
                wv0 = w0_v[pl.ds(q0, GRP)]
                wv1 = w1_v[pl.ds(q0, GRP)]
                ob = par + g * (GRP * PTS) + lanes * PTS
                for p in range(PTS):
                    a0 = plsc.load_gather(tab_v, [iv0 + p])
                    a1 = plsc.load_gather(tab_v, [iv1 + p])
                    plsc.store_scatter(out_v, [ob + p], wv0 * a0 + wv1 * a1)

            pltpu.async_copy(out_v.at[pl.ds(par, csz)],
                             dn_out.at[b, c, pl.ds(ch * csz, csz)], osem)
            return 0

        lax.fori_loop(0, NCHK, _chunk, 0)
        _drain_out(c)
        _drain_out(c)
    for ci in range(SP_PER_W):
        c = j * SP_PER_W + ci
        pltpu.sync_copy(sparse_hbm.at[b, c], srow_v)

        @plsc.parallel_loop(0, N_Q // GRP)
        def _sgroup(g):
            q0 = g * GRP
            a0 = plsc.load_gather(srow_v, [i0_v[pl.ds(q0, GRP)]])
            a1 = plsc.load_gather(srow_v, [i1_v[pl.ds(q0, GRP)]])
            sout_v[pl.ds(q0, GRP)] = (w0_v[pl.ds(q0, GRP)] * a0
                                      + w1_v[pl.ds(q0, GRP)] * a1)

        pltpu.sync_copy(sout_v, spg_out.at[b, c])


def _sc_gather(dense_flat, sparse_fea, i0, i1, w0, w1):
    fn = pl.kernel(
        _sc_gather_body,
        out_type=(jax.ShapeDtypeStruct((BS, DN_C, L_SEQ), jnp.float32),
                  jax.ShapeDtypeStruct((BS, SP_C, N_Q), jnp.float32)),
        mesh=plsc.VectorSubcoreMesh(core_axis_name="c", subcore_axis_name="s"),
        compiler_params=pltpu.CompilerParams(needs_layout_passes=False),
        scratch_types=[
            pltpu.VMEM((N_KEY * PTS,), jnp.float32),
            pltpu.VMEM((N_Q,), jnp.int32),
            pltpu.VMEM((N_Q,), jnp.int32),
            pltpu.VMEM((N_Q,), jnp.float32),
            pltpu.VMEM((N_Q,), jnp.float32),
            pltpu.VMEM((2 * QCH * PTS,), jnp.float32),
            pltpu.VMEM((N_KEY,), jnp.float32),
            pltpu.VMEM((N_Q,), jnp.float32),
            pltpu.SemaphoreType.DMA,
        ],
    )
    return fn(dense_flat, sparse_fea, i0, i1, w0, w1)


MB = 1024


def _mlp_body(w_ref, b_ref, x_ref, o_ref):
    x = x_ref[0]
    y = lax.dot_general(w_ref[...], x, (((1,), (0,)), ((), ())),
                        preferred_element_type=jnp.float32)
    y = (y + b_ref[...]) / jnp.sqrt(jnp.float32(1.0 + 1e-5))
    o_ref[0] = jax.nn.gelu(y)


def _mlp(sp_W, sp_b, spg):
    return pl.pallas_call(
        _mlp_body,
        grid=(BS, N_Q // MB),
        in_specs=[
            pl.BlockSpec((SP_C, SP_C), lambda b, m: (0, 0)),
            pl.BlockSpec((SP_C, 1), lambda b, m: (0, 0)),
            pl.BlockSpec((1, SP_C, MB), lambda b, m: (b, 0, m)),
        ],
        out_specs=pl.BlockSpec((1, SP_C, MB), lambda b, m: (b, 0, m)),
        out_shape=jax.ShapeDtypeStruct((BS, SP_C, N_Q), jnp.float32),
    )(sp_W, sp_b.reshape(SP_C, 1), spg)


CH = 8192
NCB = L_SEQ // CH


SEG = 128


def _conv_body(w_ref, b_ref, p_ref, x_ref, xm_ref, xp_ref, o_ref):
    kb = pl.program_id(1)
    x = x_ref[0]
    prev = xm_ref[0]
    nxt = xp_ref[0]
    cm = jnp.where(kb == 0, x[:, 0:1], prev[:, CH - 1:CH])
    cp = jnp.where(kb == NCB - 1, x[:, CH - 1:CH], nxt[:, 0:1])
    xm = jnp.concatenate([cm, x[:, :CH - 1]], axis=1)
    xp = jnp.concatenate([x[:, 1:], cp], axis=1)
    dot = lambda a, bb: lax.dot_general(a, bb, (((1,), (0,)), ((), ())),
                                        preferred_element_type=jnp.float32)
    w0 = w_ref[0:64]
    w1 = w_ref[64:128]
    w2 = w_ref[128:192]
    w3 = w_ref[192:256]
    ev = dot(w0, xm) + dot(w2, x)
    od = dot(w1, x) + dot(w3, xp)
    scale = 1.0 / jnp.sqrt(jnp.float32(1.0 + 1e-5))
    bias = b_ref[...]
    ev = jax.nn.gelu((ev + bias) * scale).astype(jnp.bfloat16)
    od = jax.nn.gelu((od + bias) * scale).astype(jnp.bfloat16)
    perm = p_ref[...]
    pieces = []
    for u in range(CH // SEG):
        eo = jnp.concatenate(
            [ev[:, u * SEG:(u + 1) * SEG], od[:, u * SEG:(u + 1) * SEG]], axis=1)
        pieces.append(lax.dot_general(eo, perm, (((1,), (0,)), ((), ())),
                                      preferred_element_type=jnp.float32))
    o_ref[0] = jnp.concatenate(pieces, axis=1)


def _conv(ct_W, ct_b, dn_cf):
    w_all = jnp.transpose(ct_W[:, :, ::-1], (2, 1, 0)).reshape(4 * DN_C, DN_C)
    tt = jnp.arange(2 * SEG)
    src = tt // 2 + SEG * (tt % 2)
    perm = (jnp.arange(2 * SEG)[:, None] == src[None, :]).astype(jnp.bfloat16)
    y = pl.pallas_call(
        _conv_body,
        grid=(BS, NCB),
        in_specs=[
            pl.BlockSpec((4 * DN_C, DN_C), lambda b, k: (0, 0)),
            pl.BlockSpec((DN_C, 1), lambda b, k: (0, 0)),
            pl.BlockSpec((2 * SEG, 2 * SEG), lambda b, k: (0, 0)),
            pl.BlockSpec((1, DN_C, CH), lambda b, k: (b, 0, k)),
            pl.BlockSpec((1, DN_C, CH), lambda b, k: (b, 0, jnp.maximum(k - 1, 0))),
            pl.BlockSpec((1, DN_C, CH), lambda b, k: (b, 0, jnp.minimum(k + 1, NCB - 1))),
        ],
        out_specs=pl.BlockSpec((1, DN_C, 2 * CH), lambda b, k: (b, 0, k)),
        out_shape=jax.ShapeDtypeStruct((BS, DN_C, 2 * L_SEQ), jnp.float32),
    )(w_all, ct_b.reshape(DN_C, 1), perm, dn_cf, dn_cf, dn_cf)
    return y.reshape(BS, DN_C, N_Q, 2 * L_SEQ // N_Q)


def kernel(sparse_fea, dense_fea, stk_coor, stk_coor_bef, sp_W, sp_b, ct_W, ct_b):
    i0, i1, w0, w1 = _knn(stk_coor_bef, stk_coor)
    dense_flat = dense_fea.reshape(BS, DN_C, N_KEY * PTS)
    dn_cf, spg = _sc_gather(dense_flat, sparse_fea, i0, i1, w0, w1)
    sp = _mlp(sp_W, sp_b, spg)
    y = _conv(ct_W, ct_b, dn_cf)
    return (sp, y)


# speedup vs baseline: 4.6661x; 1.1208x over previous
import functools

import jax
import jax.numpy as jnp
from jax import lax
from jax.experimental import pallas as pl
from jax.experimental.pallas import tpu as pltpu
from jax.experimental.pallas import tpu_sc as plsc

BS = 4
SP_C = 128
DN_C = 64
N_KEY = 2048
N_Q = 4096
PTS = 32
L_SEQ = N_Q * PTS

QB = 256
NQB = N_Q // QB


def _knn_body(q_ref, k_ref, i0_ref, i1_ref, w0_ref, w1_ref):
    q = q_ref[0]
    k = k_ref[0]
    d = lax.dot_general(q, k, (((1,), (1,)), ((), ())),
                        preferred_element_type=jnp.float32)
    d = (-2.0) * d
    d = d + jnp.sum(q * q, axis=1, keepdims=True)
    d = d + jnp.sum(k * k, axis=1)[None, :]
    m1 = jnp.min(d, axis=1)
    a1 = jnp.argmin(d, axis=1).astype(jnp.int32)
    it = lax.broadcasted_iota(jnp.int32, d.shape, 1)
    d2 = jnp.where(it == a1[:, None], jnp.float32(jnp.inf), d)
    m2 = jnp.min(d2, axis=1)
    a2 = jnp.argmin(d2, axis=1).astype(jnp.int32)
    r1 = 1.0 / (m1 + 1e-8)
    r2 = 1.0 / (m2 + 1e-8)
    s = r1 + r2
    i0_ref[0, 0] = a1
    i1_ref[0, 0] = a2
    w0_ref[0, 0] = r1 / s
    w1_ref[0, 0] = r2 / s


def _knn(stk_coor_bef, stk_coor):
    out3 = lambda dt: jax.ShapeDtypeStruct((BS * NQB, 1, QB), dt)
    res = pl.pallas_call(
        _knn_body,
        grid=(BS, NQB),
        in_specs=[
            pl.BlockSpec((1, QB, 64), lambda b, qi: (b, qi, 0)),
            pl.BlockSpec((1, N_KEY, 64), lambda b, qi: (b, 0, 0)),
        ],
        out_specs=[pl.BlockSpec((1, 1, QB), lambda b, qi: (b * NQB + qi, 0, 0))] * 4,
        out_shape=[out3(jnp.int32), out3(jnp.int32), out3(jnp.float32), out3(jnp.float32)],
    )(stk_coor_bef, stk_coor)
    return tuple(r.reshape(BS, N_Q) for r in res)


NC, NS = 2, 16
NW = NC * NS
W_PER_B = NW // BS
DN_PER_W = DN_C // W_PER_B
SP_PER_W = SP_C // W_PER_B
QCH = 512
NCHK = N_Q // QCH
GRP = 16
NGRP = QCH // GRP


def _sc_gather_body(dense_hbm, sparse_hbm, i0_hbm, i1_hbm, w0_hbm, w1_hbm,
                    dn_out, spg_out,
                    tab_v, i0_v, i1_v, w0_v, w1_v, out_v, srow_v, sout_v,
                    osem):
    wid = lax.axis_index("s") * NC + lax.axis_index("c")
    b = wid // W_PER_B
    j = wid % W_PER_B
    pltpu.sync_copy(i0_hbm.at[b], i0_v)
    pltpu.sync_copy(i1_hbm.at[b], i1_v)
    pltpu.sync_copy(w0_hbm.at[b], w0_v)
    pltpu.sync_copy(w1_hbm.at[b], w1_v)
    lanes = lax.iota(jnp.int32, 16)
    csz = QCH * PTS

    def _drain_out(c):
        pltpu.make_async_copy(
            out_v.at[pl.ds(0, csz)], dn_out.at[b, c, pl.ds(0, csz)], osem
        ).wait()

    for ci in range(DN_PER_W):
        c = j * DN_PER_W + ci
        pltpu.sync_copy(dense_hbm.at[b, c], tab_v)

        def _chunk(ch, _, c=c):
            par = (ch % 2) * csz

            @pl.when(ch >= 2)
            def _():
                _drain_out(c)

            @plsc.parallel_loop(0, NGRP)
            def _group(g):
                q0 = ch * QCH + g * GRP
                iv0 = i0_v[pl.ds(q0, GRP)] * PTS
                iv1 = i1_v[pl.ds(q0, GRP)] * PTS
                wv0 = w0_v[pl.ds(q0, GRP)]
                wv1 = w1_v[pl.ds(q0, GRP)]
                ob = par + g * (GRP * PTS) + lanes * PTS

                @plsc.parallel_loop(0, PTS, unroll=8)
                def _pt(p):
                    a0 = plsc.load_gather(tab_v, [iv0 + p])
                    a1 = plsc.load_gather(tab_v, [iv1 + p])
                    plsc.store_scatter(out_v, [ob + p], wv0 * a0 + wv1 * a1)

            pltpu.async_copy(out_v.at[pl.ds(par, csz)],
                             dn_out.at[b, c, pl.ds(ch * csz, csz)], osem)
            return 0

        lax.fori_loop(0, NCHK, _chunk, 0)
        _drain_out(c)
        _drain_out(c)
    for ci in range(SP_PER_W):
        c = j * SP_PER_W + ci
        pltpu.sync_copy(sparse_hbm.at[b, c], srow_v)

        @plsc.parallel_loop(0, N_Q // GRP, unroll=4)
        def _sgroup(g):
            q0 = g * GRP
            a0 = plsc.load_gather(srow_v, [i0_v[pl.ds(q0, GRP)]])
            a1 = plsc.load_gather(srow_v, [i1_v[pl.ds(q0, GRP)]])
            sout_v[pl.ds(q0, GRP)] = (w0_v[pl.ds(q0, GRP)] * a0
                                      + w1_v[pl.ds(q0, GRP)] * a1)

        pltpu.sync_copy(sout_v, spg_out.at[b, c])


def _sc_gather(dense_flat, sparse_fea, i0, i1, w0, w1):
    fn = pl.kernel(
        _sc_gather_body,
        out_type=(jax.ShapeDtypeStruct((BS, DN_C, L_SEQ), jnp.float32),
                  jax.ShapeDtypeStruct((BS, SP_C, N_Q), jnp.float32)),
        mesh=plsc.VectorSubcoreMesh(core_axis_name="c", subcore_axis_name="s"),
        compiler_params=pltpu.CompilerParams(needs_layout_passes=False),
        scratch_types=[
            pltpu.VMEM((N_KEY * PTS,), jnp.float32),
            pltpu.VMEM((N_Q,), jnp.int32),
            pltpu.VMEM((N_Q,), jnp.int32),
            pltpu.VMEM((N_Q,), jnp.float32),
            pltpu.VMEM((N_Q,), jnp.float32),
            pltpu.VMEM((2 * QCH * PTS,), jnp.float32),
            pltpu.VMEM((N_KEY,), jnp.float32),
            pltpu.VMEM((N_Q,), jnp.float32),
            pltpu.SemaphoreType.DMA,
        ],
    )
    return fn(dense_flat, sparse_fea, i0, i1, w0, w1)


MB = 1024


def _mlp_body(w_ref, b_ref, x_ref, o_ref):
    x = x_ref[0]
    y = lax.dot_general(w_ref[...], x, (((1,), (0,)), ((), ())),
                        preferred_element_type=jnp.float32)
    y = (y + b_ref[...]) / jnp.sqrt(jnp.float32(1.0 + 1e-5))
    o_ref[0] = jax.nn.gelu(y)


def _mlp(sp_W, sp_b, spg):
    return pl.pallas_call(
        _mlp_body,
        grid=(BS, N_Q // MB),
        in_specs=[
            pl.BlockSpec((SP_C, SP_C), lambda b, m: (0, 0)),
            pl.BlockSpec((SP_C, 1), lambda b, m: (0, 0)),
            pl.BlockSpec((1, SP_C, MB), lambda b, m: (b, 0, m)),
        ],
        out_specs=pl.BlockSpec((1, SP_C, MB), lambda b, m: (b, 0, m)),
        out_shape=jax.ShapeDtypeStruct((BS, SP_C, N_Q), jnp.float32),
    )(sp_W, sp_b.reshape(SP_C, 1), spg)


CH = 8192
NCB = L_SEQ // CH


SEG = 128


def _conv_body(w_ref, b_ref, p_ref, x_ref, xm_ref, xp_ref, o_ref):
    kb = pl.program_id(1)
    x = x_ref[0]
    prev = xm_ref[0]
    nxt = xp_ref[0]
    cm = jnp.where(kb == 0, x[:, 0:1], prev[:, CH - 1:CH])
    cp = jnp.where(kb == NCB - 1, x[:, CH - 1:CH], nxt[:, 0:1])
    xm = jnp.concatenate([cm, x[:, :CH - 1]], axis=1)
    xp = jnp.concatenate([x[:, 1:], cp], axis=1)
    dot = lambda a, bb: lax.dot_general(a, bb, (((1,), (0,)), ((), ())),
                                        preferred_element_type=jnp.float32)
    w0 = w_ref[0:64]
    w1 = w_ref[64:128]
    w2 = w_ref[128:192]
    w3 = w_ref[192:256]
    ev = dot(w0, xm) + dot(w2, x)
    od = dot(w1, x) + dot(w3, xp)
    scale = 1.0 / jnp.sqrt(jnp.float32(1.0 + 1e-5))
    bias = b_ref[...]
    ev = jax.nn.gelu((ev + bias) * scale).astype(jnp.bfloat16)
    od = jax.nn.gelu((od + bias) * scale).astype(jnp.bfloat16)
    perm = p_ref[...]
    pieces = []
    for u in range(CH // SEG):
        eo = jnp.concatenate(
            [ev[:, u * SEG:(u + 1) * SEG], od[:, u * SEG:(u + 1) * SEG]], axis=1)
        pieces.append(lax.dot_general(eo, perm, (((1,), (0,)), ((), ())),
                                      preferred_element_type=jnp.float32))
    o_ref[0] = jnp.concatenate(pieces, axis=1)


def _conv(ct_W, ct_b, dn_cf):
    w_all = jnp.transpose(ct_W[:, :, ::-1], (2, 1, 0)).reshape(4 * DN_C, DN_C)
    tt = jnp.arange(2 * SEG)
    src = tt // 2 + SEG * (tt % 2)
    perm = (jnp.arange(2 * SEG)[:, None] == src[None, :]).astype(jnp.bfloat16)
    y = pl.pallas_call(
        _conv_body,
        grid=(BS, NCB),
        in_specs=[
            pl.BlockSpec((4 * DN_C, DN_C), lambda b, k: (0, 0)),
            pl.BlockSpec((DN_C, 1), lambda b, k: (0, 0)),
            pl.BlockSpec((2 * SEG, 2 * SEG), lambda b, k: (0, 0)),
            pl.BlockSpec((1, DN_C, CH), lambda b, k: (b, 0, k)),
            pl.BlockSpec((1, DN_C, CH), lambda b, k: (b, 0, jnp.maximum(k - 1, 0))),
            pl.BlockSpec((1, DN_C, CH), lambda b, k: (b, 0, jnp.minimum(k + 1, NCB - 1))),
        ],
        out_specs=pl.BlockSpec((1, DN_C, 2 * CH), lambda b, k: (b, 0, k)),
        out_shape=jax.ShapeDtypeStruct((BS, DN_C, 2 * L_SEQ), jnp.float32),
    )(w_all, ct_b.reshape(DN_C, 1), perm, dn_cf, dn_cf, dn_cf)
    return y.reshape(BS, DN_C, N_Q, 2 * L_SEQ // N_Q)


def kernel(sparse_fea, dense_fea, stk_coor, stk_coor_bef, sp_W, sp_b, ct_W, ct_b):
    i0, i1, w0, w1 = _knn(stk_coor_bef, stk_coor)
    dense_flat = dense_fea.reshape(BS, DN_C, N_KEY * PTS)
    dn_cf, spg = _sc_gather(dense_flat, sparse_fea, i0, i1, w0, w1)
    sp = _mlp(sp_W, sp_b, spg)
    y = _conv(ct_W, ct_b, dn_cf)
    return (sp, y)


# speedup vs baseline: 7.4919x; 1.6056x over previous
import functools

import jax
import jax.numpy as jnp
from jax import lax
from jax.experimental import pallas as pl
from jax.experimental.pallas import tpu as pltpu
from jax.experimental.pallas import tpu_sc as plsc

BS = 4
SP_C = 128
DN_C = 64
N_KEY = 2048
N_Q = 4096
PTS = 32
L_SEQ = N_Q * PTS

QB = 256
NQB = N_Q // QB


def _knn_body(q_ref, k_ref, i0_ref, i1_ref, w0_ref, w1_ref):
    q = q_ref[0]
    k = k_ref[0]
    d = lax.dot_general(q, k, (((1,), (1,)), ((), ())),
                        preferred_element_type=jnp.float32)
    d = (-2.0) * d
    d = d + jnp.sum(q * q, axis=1, keepdims=True)
    d = d + jnp.sum(k * k, axis=1)[None, :]
    m1 = jnp.min(d, axis=1)
    a1 = jnp.argmin(d, axis=1).astype(jnp.int32)
    it = lax.broadcasted_iota(jnp.int32, d.shape, 1)
    d2 = jnp.where(it == a1[:, None], jnp.float32(jnp.inf), d)
    m2 = jnp.min(d2, axis=1)
    a2 = jnp.argmin(d2, axis=1).astype(jnp.int32)
    r1 = 1.0 / (m1 + 1e-8)
    r2 = 1.0 / (m2 + 1e-8)
    s = r1 + r2
    i0_ref[0, 0] = a1
    i1_ref[0, 0] = a2
    w0_ref[0, 0] = r1 / s
    w1_ref[0, 0] = r2 / s


def _knn(stk_coor_bef, stk_coor):
    out3 = lambda dt: jax.ShapeDtypeStruct((BS * NQB, 1, QB), dt)
    res = pl.pallas_call(
        _knn_body,
        grid=(BS, NQB),
        in_specs=[
            pl.BlockSpec((1, QB, 64), lambda b, qi: (b, qi, 0)),
            pl.BlockSpec((1, N_KEY, 64), lambda b, qi: (b, 0, 0)),
        ],
        out_specs=[pl.BlockSpec((1, 1, QB), lambda b, qi: (b * NQB + qi, 0, 0))] * 4,
        out_shape=[out3(jnp.int32), out3(jnp.int32), out3(jnp.float32), out3(jnp.float32)],
    )(stk_coor_bef, stk_coor)
    return tuple(r.reshape(BS, N_Q) for r in res)


NC, NS = 2, 16
NW = NC * NS
W_PER_B = NW // BS
DN_PER_W = DN_C // W_PER_B
SP_PER_W = SP_C // W_PER_B
QCH = 512
NCHK = N_Q // QCH
GRP = 16
NGRP = QCH // GRP


def _sc_gather_body(dense_hbm, sparse_hbm, i0_hbm, i1_hbm, w0_hbm, w1_hbm,
                    dn_out, spg_out,
                    tab_v, i0_v, i1_v, w0_v, w1_v, out_v, srow_v, sout_v,
                    osem):
    wid = lax.axis_index("s") * NC + lax.axis_index("c")
    b = wid // W_PER_B
    j = wid % W_PER_B
    pltpu.sync_copy(i0_hbm.at[b], i0_v)
    pltpu.sync_copy(i1_hbm.at[b], i1_v)
    pltpu.sync_copy(w0_hbm.at[b], w0_v)
    pltpu.sync_copy(w1_hbm.at[b], w1_v)
    lanes = lax.iota(jnp.int32, 16)
    csz = QCH * PTS

    def _drain_out(c):
        pltpu.make_async_copy(
            out_v.at[pl.ds(0, csz)], dn_out.at[b, c, pl.ds(0, csz)], osem
        ).wait()

    for ci in range(DN_PER_W):
        c = j * DN_PER_W + ci
        pltpu.sync_copy(dense_hbm.at[b, c], tab_v)

        def _chunk(ch, _, c=c):
            par = (ch % 2) * csz

            @pl.when(ch >= 2)
            def _():
                _drain_out(c)

            @plsc.parallel_loop(0, NGRP)
            def _group(g):
                q0 = ch * QCH + g * GRP
                iv0 = i0_v[pl.ds(q0, GRP)] * PTS
                iv1 = i1_v[pl.ds(q0, GRP)] * PTS
                wv0 = w0_v[pl.ds(q0, GRP)]
                wv1 = w1_v[pl.ds(q0, GRP)]
                ob = par + g * (GRP * PTS) + lanes * PTS

                @plsc.parallel_loop(0, PTS, unroll=8)
                def _pt(p):
                    prot = jnp.bitwise_and(lanes + p, PTS - 1)
                    a0 = plsc.load_gather(tab_v, [iv0 + prot])
                    a1 = plsc.load_gather(tab_v, [iv1 + prot])
                    plsc.store_scatter(out_v, [ob + prot], wv0 * a0 + wv1 * a1)

            pltpu.async_copy(out_v.at[pl.ds(par, csz)],
                             dn_out.at[b, c, pl.ds(ch * csz, csz)], osem)
            return 0

        lax.fori_loop(0, NCHK, _chunk, 0)
        _drain_out(c)
        _drain_out(c)
    for ci in range(SP_PER_W):
        c = j * SP_PER_W + ci
        pltpu.sync_copy(sparse_hbm.at[b, c], srow_v)

        @plsc.parallel_loop(0, N_Q // GRP, unroll=4)
        def _sgroup(g):
            q0 = g * GRP
            a0 = plsc.load_gather(srow_v, [i0_v[pl.ds(q0, GRP)]])
            a1 = plsc.load_gather(srow_v, [i1_v[pl.ds(q0, GRP)]])
            sout_v[pl.ds(q0, GRP)] = (w0_v[pl.ds(q0, GRP)] * a0
                                      + w1_v[pl.ds(q0, GRP)] * a1)

        pltpu.sync_copy(sout_v, spg_out.at[b, c])


def _sc_gather(dense_flat, sparse_fea, i0, i1, w0, w1):
    fn = pl.kernel(
        _sc_gather_body,
        out_type=(jax.ShapeDtypeStruct((BS, DN_C, L_SEQ), jnp.float32),
                  jax.ShapeDtypeStruct((BS, SP_C, N_Q), jnp.float32)),
        mesh=plsc.VectorSubcoreMesh(core_axis_name="c", subcore_axis_name="s"),
        compiler_params=pltpu.CompilerParams(needs_layout_passes=False),
        scratch_types=[
            pltpu.VMEM((N_KEY * PTS,), jnp.float32),
            pltpu.VMEM((N_Q,), jnp.int32),
            pltpu.VMEM((N_Q,), jnp.int32),
            pltpu.VMEM((N_Q,), jnp.float32),
            pltpu.VMEM((N_Q,), jnp.float32),
            pltpu.VMEM((2 * QCH * PTS,), jnp.float32),
            pltpu.VMEM((N_KEY,), jnp.float32),
            pltpu.VMEM((N_Q,), jnp.float32),
            pltpu.SemaphoreType.DMA,
        ],
    )
    return fn(dense_flat, sparse_fea, i0, i1, w0, w1)


MB = 1024


def _mlp_body(w_ref, b_ref, x_ref, o_ref):
    x = x_ref[0]
    y = lax.dot_general(w_ref[...], x, (((1,), (0,)), ((), ())),
                        preferred_element_type=jnp.float32)
    y = (y + b_ref[...]) / jnp.sqrt(jnp.float32(1.0 + 1e-5))
    o_ref[0] = jax.nn.gelu(y)


def _mlp(sp_W, sp_b, spg):
    return pl.pallas_call(
        _mlp_body,
        grid=(BS, N_Q // MB),
        in_specs=[
            pl.BlockSpec((SP_C, SP_C), lambda b, m: (0, 0)),
            pl.BlockSpec((SP_C, 1), lambda b, m: (0, 0)),
            pl.BlockSpec((1, SP_C, MB), lambda b, m: (b, 0, m)),
        ],
        out_specs=pl.BlockSpec((1, SP_C, MB), lambda b, m: (b, 0, m)),
        out_shape=jax.ShapeDtypeStruct((BS, SP_C, N_Q), jnp.float32),
    )(sp_W, sp_b.reshape(SP_C, 1), spg)


CH = 8192
NCB = L_SEQ // CH


SEG = 128


def _conv_body(w_ref, b_ref, p_ref, x_ref, xm_ref, xp_ref, o_ref):
    kb = pl.program_id(1)
    x = x_ref[0]
    prev = xm_ref[0]
    nxt = xp_ref[0]
    cm = jnp.where(kb == 0, x[:, 0:1], prev[:, CH - 1:CH])
    cp = jnp.where(kb == NCB - 1, x[:, CH - 1:CH], nxt[:, 0:1])
    xm = jnp.concatenate([cm, x[:, :CH - 1]], axis=1)
    xp = jnp.concatenate([x[:, 1:], cp], axis=1)
    dot = lambda a, bb: lax.dot_general(a, bb, (((1,), (0,)), ((), ())),
                                        preferred_element_type=jnp.float32)
    w0 = w_ref[0:64]
    w1 = w_ref[64:128]
    w2 = w_ref[128:192]
    w3 = w_ref[192:256]
    ev = dot(w0, xm) + dot(w2, x)
    od = dot(w1, x) + dot(w3, xp)
    scale = 1.0 / jnp.sqrt(jnp.float32(1.0 + 1e-5))
    bias = b_ref[...]
    ev = jax.nn.gelu((ev + bias) * scale).astype(jnp.bfloat16)
    od = jax.nn.gelu((od + bias) * scale).astype(jnp.bfloat16)
    perm = p_ref[...]
    pieces = []
    for u in range(CH // SEG):
        eo = jnp.concatenate(
            [ev[:, u * SEG:(u + 1) * SEG], od[:, u * SEG:(u + 1) * SEG]], axis=1)
        pieces.append(lax.dot_general(eo, perm, (((1,), (0,)), ((), ())),
                                      preferred_element_type=jnp.float32))
    o_ref[0] = jnp.concatenate(pieces, axis=1)


def _conv(ct_W, ct_b, dn_cf):
    w_all = jnp.transpose(ct_W[:, :, ::-1], (2, 1, 0)).reshape(4 * DN_C, DN_C)
    tt = jnp.arange(2 * SEG)
    src = tt // 2 + SEG * (tt % 2)
    perm = (jnp.arange(2 * SEG)[:, None] == src[None, :]).astype(jnp.bfloat16)
    y = pl.pallas_call(
        _conv_body,
        grid=(BS, NCB),
        in_specs=[
            pl.BlockSpec((4 * DN_C, DN_C), lambda b, k: (0, 0)),
            pl.BlockSpec((DN_C, 1), lambda b, k: (0, 0)),
            pl.BlockSpec((2 * SEG, 2 * SEG), lambda b, k: (0, 0)),
            pl.BlockSpec((1, DN_C, CH), lambda b, k: (b, 0, k)),
            pl.BlockSpec((1, DN_C, CH), lambda b, k: (b, 0, jnp.maximum(k - 1, 0))),
            pl.BlockSpec((1, DN_C, CH), lambda b, k: (b, 0, jnp.minimum(k + 1, NCB - 1))),
        ],
        out_specs=pl.BlockSpec((1, DN_C, 2 * CH), lambda b, k: (b, 0, k)),
        out_shape=jax.ShapeDtypeStruct((BS, DN_C, 2 * L_SEQ), jnp.float32),
    )(w_all, ct_b.reshape(DN_C, 1), perm, dn_cf, dn_cf, dn_cf)
    return y.reshape(BS, DN_C, N_Q, 2 * L_SEQ // N_Q)


def kernel(sparse_fea, dense_fea, stk_coor, stk_coor_bef, sp_W, sp_b, ct_W, ct_b):
    i0, i1, w0, w1 = _knn(stk_coor_bef, stk_coor)
    dense_flat = dense_fea.reshape(BS, DN_C, N_KEY * PTS)
    dn_cf, spg = _sc_gather(dense_flat, sparse_fea, i0, i1, w0, w1)
    sp = _mlp(sp_W, sp_b, spg)
    y = _conv(ct_W, ct_b, dn_cf)
    return (sp, y)


# speedup vs baseline: 9.0944x; 1.2139x over previous
import functools

import jax
import jax.numpy as jnp
from jax import lax
from jax.experimental import pallas as pl
from jax.experimental.pallas import tpu as pltpu
from jax.experimental.pallas import tpu_sc as plsc

BS = 4
SP_C = 128
DN_C = 64
N_KEY = 2048
N_Q = 4096
PTS = 32
L_SEQ = N_Q * PTS

QB = 256
NQB = N_Q // QB


def _knn_body(q_ref, k_ref, i0_ref, i1_ref, w0_ref, w1_ref):
    q = q_ref[0]
    k = k_ref[0]
    d = lax.dot_general(q, k, (((1,), (1,)), ((), ())),
                        preferred_element_type=jnp.float32)
    d = (-2.0) * d
    d = d + jnp.sum(q * q, axis=1, keepdims=True)
    d = d + jnp.sum(k * k, axis=1)[None, :]
    m1 = jnp.min(d, axis=1)
    a1 = jnp.argmin(d, axis=1).astype(jnp.int32)
    it = lax.broadcasted_iota(jnp.int32, d.shape, 1)
    d2 = jnp.where(it == a1[:, None], jnp.float32(jnp.inf), d)
    m2 = jnp.min(d2, axis=1)
    a2 = jnp.argmin(d2, axis=1).astype(jnp.int32)
    r1 = 1.0 / (m1 + 1e-8)
    r2 = 1.0 / (m2 + 1e-8)
    s = r1 + r2
    i0_ref[0, 0] = a1
    i1_ref[0, 0] = a2
    w0_ref[0, 0] = r1 / s
    w1_ref[0, 0] = r2 / s


def _knn(stk_coor_bef, stk_coor):
    out3 = lambda dt: jax.ShapeDtypeStruct((BS * NQB, 1, QB), dt)
    res = pl.pallas_call(
        _knn_body,
        grid=(BS, NQB),
        in_specs=[
            pl.BlockSpec((1, QB, 64), lambda b, qi: (b, qi, 0)),
            pl.BlockSpec((1, N_KEY, 64), lambda b, qi: (b, 0, 0)),
        ],
        out_specs=[pl.BlockSpec((1, 1, QB), lambda b, qi: (b * NQB + qi, 0, 0))] * 4,
        out_shape=[out3(jnp.int32), out3(jnp.int32), out3(jnp.float32), out3(jnp.float32)],
    )(stk_coor_bef, stk_coor)
    return tuple(r.reshape(BS, N_Q) for r in res)


NC, NS = 2, 16
NW = NC * NS
W_PER_B = NW // BS
DN_PER_W = DN_C // W_PER_B
SP_PER_W = SP_C // W_PER_B
QCH = 512
NCHK = N_Q // QCH
GRP = 16
NGRP = QCH // GRP


def _sc_gather_body(dense_hbm, sparse_hbm, i0_hbm, i1_hbm, w0_hbm, w1_hbm,
                    dn_out, spg_out,
                    tab_v, i0_v, i1_v, w0_v, w1_v, out_v, srow_v, sout_v,
                    osem):
    wid = lax.axis_index("s") * NC + lax.axis_index("c")
    b = wid // W_PER_B
    j = wid % W_PER_B
    pltpu.sync_copy(i0_hbm.at[b], i0_v)
    pltpu.sync_copy(i1_hbm.at[b], i1_v)
    pltpu.sync_copy(w0_hbm.at[b], w0_v)
    pltpu.sync_copy(w1_hbm.at[b], w1_v)
    lanes = lax.iota(jnp.int32, 16)
    csz = QCH * PTS

    def _drain_out(c):
        pltpu.make_async_copy(
            out_v.at[pl.ds(0, csz)], dn_out.at[b, c, pl.ds(0, csz)], osem
        ).wait()

    for ci in range(DN_PER_W):
        c = j * DN_PER_W + ci
        pltpu.sync_copy(dense_hbm.at[b, c], tab_v)

        def _chunk(ch, _, c=c):
            par = (ch % 2) * csz

            @pl.when(ch >= 2)
            def _():
                _drain_out(c)

            @plsc.parallel_loop(0, NGRP)
            def _group(g):
                q0 = ch * QCH + g * GRP
                iv0 = i0_v[pl.ds(q0, GRP)] * PTS
                iv1 = i1_v[pl.ds(q0, GRP)] * PTS
                wv0 = w0_v[pl.ds(q0, GRP)]
                wv1 = w1_v[pl.ds(q0, GRP)]
                ob = par + g * (GRP * PTS) + lanes * PTS

                @plsc.parallel_loop(0, PTS, unroll=8)
                def _pt(p):
                    prot = jnp.bitwise_and(lanes + p, PTS - 1)
                    a0 = plsc.load_gather(tab_v, [iv0 + prot])
                    a1 = plsc.load_gather(tab_v, [iv1 + prot])
                    plsc.store_scatter(out_v, [ob + prot], wv0 * a0 + wv1 * a1)

            pltpu.async_copy(out_v.at[pl.ds(par, csz)],
                             dn_out.at[b, c, pl.ds(ch * csz, csz)], osem)
            return 0

        lax.fori_loop(0, NCHK, _chunk, 0)
        _drain_out(c)
        _drain_out(c)
    for ci in range(SP_PER_W):
        c = j * SP_PER_W + ci
        pltpu.sync_copy(sparse_hbm.at[b, c], srow_v)

        @plsc.parallel_loop(0, N_Q // GRP, unroll=4)
        def _sgroup(g):
            q0 = g * GRP
            a0 = plsc.load_gather(srow_v, [i0_v[pl.ds(q0, GRP)]])
            a1 = plsc.load_gather(srow_v, [i1_v[pl.ds(q0, GRP)]])
            sout_v[pl.ds(q0, GRP)] = (w0_v[pl.ds(q0, GRP)] * a0
                                      + w1_v[pl.ds(q0, GRP)] * a1)

        pltpu.sync_copy(sout_v, spg_out.at[b, c])


def _sc_gather(dense_flat, sparse_fea, i0, i1, w0, w1):
    fn = pl.kernel(
        _sc_gather_body,
        out_type=(jax.ShapeDtypeStruct((BS, DN_C, L_SEQ), jnp.float32),
                  jax.ShapeDtypeStruct((BS, SP_C, N_Q), jnp.float32)),
        mesh=plsc.VectorSubcoreMesh(core_axis_name="c", subcore_axis_name="s"),
        compiler_params=pltpu.CompilerParams(needs_layout_passes=False),
        scratch_types=[
            pltpu.VMEM((N_KEY * PTS,), jnp.float32),
            pltpu.VMEM((N_Q,), jnp.int32),
            pltpu.VMEM((N_Q,), jnp.int32),
            pltpu.VMEM((N_Q,), jnp.float32),
            pltpu.VMEM((N_Q,), jnp.float32),
            pltpu.VMEM((2 * QCH * PTS,), jnp.float32),
            pltpu.VMEM((N_KEY,), jnp.float32),
            pltpu.VMEM((N_Q,), jnp.float32),
            pltpu.SemaphoreType.DMA,
        ],
    )
    return fn(dense_flat, sparse_fea, i0, i1, w0, w1)


MB = 1024


def _mlp_body(w_ref, b_ref, x_ref, o_ref):
    x = x_ref[0]
    y = lax.dot_general(w_ref[...], x, (((1,), (0,)), ((), ())),
                        preferred_element_type=jnp.float32)
    y = (y + b_ref[...]) / jnp.sqrt(jnp.float32(1.0 + 1e-5))
    o_ref[0] = jax.nn.gelu(y)


def _mlp(sp_W, sp_b, spg):
    return pl.pallas_call(
        _mlp_body,
        grid=(BS, N_Q // MB),
        in_specs=[
            pl.BlockSpec((SP_C, SP_C), lambda b, m: (0, 0)),
            pl.BlockSpec((SP_C, 1), lambda b, m: (0, 0)),
            pl.BlockSpec((1, SP_C, MB), lambda b, m: (b, 0, m)),
        ],
        out_specs=pl.BlockSpec((1, SP_C, MB), lambda b, m: (b, 0, m)),
        out_shape=jax.ShapeDtypeStruct((BS, SP_C, N_Q), jnp.float32),
    )(sp_W, sp_b.reshape(SP_C, 1), spg)


CH = 8192
NCB = L_SEQ // CH


SEG = 128


def _conv_body(w_ref, b_ref, p_ref, x_ref, xm_ref, xp_ref, o_ref):
    kb = pl.program_id(1)
    x = x_ref[0]
    prev = xm_ref[0]
    nxt = xp_ref[0]
    cm = jnp.where(kb == 0, x[:, 0:1], prev[:, 127:128])
    cp = jnp.where(kb == NCB - 1, x[:, CH - 1:CH], nxt[:, 0:1])
    xm = jnp.concatenate([cm, x[:, :CH - 1]], axis=1)
    xp = jnp.concatenate([x[:, 1:], cp], axis=1)
    dot = lambda a, bb: lax.dot_general(a, bb, (((1,), (0,)), ((), ())),
                                        preferred_element_type=jnp.float32)
    w0 = w_ref[0:64]
    w1 = w_ref[64:128]
    w2 = w_ref[128:192]
    w3 = w_ref[192:256]
    ev = dot(w0, xm) + dot(w2, x)
    od = dot(w1, x) + dot(w3, xp)
    scale = 1.0 / jnp.sqrt(jnp.float32(1.0 + 1e-5))
    bias = b_ref[...]
    ev = jax.nn.gelu((ev + bias) * scale).astype(jnp.bfloat16)
    od = jax.nn.gelu((od + bias) * scale).astype(jnp.bfloat16)
    perm = p_ref[...]
    pieces = []
    for u in range(CH // SEG):
        eo = jnp.concatenate(
            [ev[:, u * SEG:(u + 1) * SEG], od[:, u * SEG:(u + 1) * SEG]], axis=1)
        pieces.append(lax.dot_general(eo, perm, (((1,), (0,)), ((), ())),
                                      preferred_element_type=jnp.float32))
    y = jnp.concatenate(pieces, axis=1)
    o_ref[0] = y.reshape(DN_C, 2 * CH // 64, 64)


def _conv(ct_W, ct_b, dn_cf):
    w_all = jnp.transpose(ct_W[:, :, ::-1], (2, 1, 0)).reshape(4 * DN_C, DN_C)
    tt = jnp.arange(2 * SEG)
    src = tt // 2 + SEG * (tt % 2)
    perm = (jnp.arange(2 * SEG)[:, None] == src[None, :]).astype(jnp.bfloat16)
    nhalo = CH // 128
    y = pl.pallas_call(
        _conv_body,
        grid=(BS, NCB),
        in_specs=[
            pl.BlockSpec((4 * DN_C, DN_C), lambda b, k: (0, 0)),
            pl.BlockSpec((DN_C, 1), lambda b, k: (0, 0)),
            pl.BlockSpec((2 * SEG, 2 * SEG), lambda b, k: (0, 0)),
            pl.BlockSpec((1, DN_C, CH), lambda b, k: (b, 0, k)),
            pl.BlockSpec((1, DN_C, 128),
                         lambda b, k: (b, 0, jnp.maximum(k * nhalo - 1, 0))),
            pl.BlockSpec((1, DN_C, 128),
                         lambda b, k: (b, 0, jnp.minimum((k + 1) * nhalo,
                                                         L_SEQ // 128 - 1))),
        ],
        out_specs=pl.BlockSpec((1, DN_C, 2 * CH // 64, 64), lambda b, k: (b, 0, k, 0)),
        out_shape=jax.ShapeDtypeStruct((BS, DN_C, N_Q, 2 * L_SEQ // N_Q), jnp.float32),
    )(w_all, ct_b.reshape(DN_C, 1), perm, dn_cf, dn_cf, dn_cf)
    return y


def kernel(sparse_fea, dense_fea, stk_coor, stk_coor_bef, sp_W, sp_b, ct_W, ct_b):
    i0, i1, w0, w1 = _knn(stk_coor_bef, stk_coor)
    dense_flat = dense_fea.reshape(BS, DN_C, N_KEY * PTS)
    dn_cf, spg = _sc_gather(dense_flat, sparse_fea, i0, i1, w0, w1)
    sp = _mlp(sp_W, sp_b, spg)
    y = _conv(ct_W, ct_b, dn_cf)
    return (sp, y)


# speedup vs baseline: 9.0971x; 1.0003x over previous
import jax
import jax.numpy as jnp
from jax import lax
from jax.experimental import pallas as pl
from jax.experimental.pallas import tpu as pltpu
from jax.experimental.pallas import tpu_sc as plsc

BS = 4
SP_C = 128
DN_C = 64
N_KEY = 2048
N_Q = 4096
PTS = 32
L_SEQ = N_Q * PTS

QB = 256
NQB = N_Q // QB


def _knn_body(q_ref, k_ref, i0_ref, i1_ref, w0_ref, w1_ref):
    q = q_ref[0]
    k = k_ref[0]
    d = lax.dot_general(q, k, (((1,), (1,)), ((), ())),
                        preferred_element_type=jnp.float32)
    d = (-2.0) * d
    d = d + jnp.sum(q * q, axis=1, keepdims=True)
    d = d + jnp.sum(k * k, axis=1)[None, :]
    m1 = jnp.min(d, axis=1)
    a1 = jnp.argmin(d, axis=1).astype(jnp.int32)
    it = lax.broadcasted_iota(jnp.int32, d.shape, 1)
    d2 = jnp.where(it == a1[:, None], jnp.float32(jnp.inf), d)
    m2 = jnp.min(d2, axis=1)
    a2 = jnp.argmin(d2, axis=1).astype(jnp.int32)
    r1 = 1.0 / (m1 + 1e-8)
    r2 = 1.0 / (m2 + 1e-8)
    s = r1 + r2
    i0_ref[0, 0] = a1
    i1_ref[0, 0] = a2
    w0_ref[0, 0] = r1 / s
    w1_ref[0, 0] = r2 / s


def _knn(stk_coor_bef, stk_coor):
    out3 = lambda dt: jax.ShapeDtypeStruct((BS * NQB, 1, QB), dt)
    res = pl.pallas_call(
        _knn_body,
        grid=(BS, NQB),
        in_specs=[
            pl.BlockSpec((1, QB, 64), lambda b, qi: (b, qi, 0)),
            pl.BlockSpec((1, N_KEY, 64), lambda b, qi: (b, 0, 0)),
        ],
        out_specs=[pl.BlockSpec((1, 1, QB), lambda b, qi: (b * NQB + qi, 0, 0))] * 4,
        out_shape=[out3(jnp.int32), out3(jnp.int32), out3(jnp.float32), out3(jnp.float32)],
    )(stk_coor_bef, stk_coor)
    return tuple(r.reshape(BS, N_Q) for r in res)


NC, NS = 2, 16
NW = NC * NS
W_PER_B = NW // BS
DN_PER_W = DN_C // W_PER_B
SP_PER_W = SP_C // W_PER_B
QCH = 512
NCHK = N_Q // QCH
GRP = 16
NGRP = QCH // GRP


def _sc_gather_body(dense_hbm, sparse_hbm, i0_hbm, i1_hbm, w0_hbm, w1_hbm,
                    dn_out, spg_out,
                    tab_v, i0_v, i1_v, w0_v, w1_v, out_v, srow_v, sout_v,
                    osem):
    wid = lax.axis_index("s") * NC + lax.axis_index("c")
    b = wid // W_PER_B
    j = wid % W_PER_B
    pltpu.sync_copy(i0_hbm.at[b], i0_v)
    pltpu.sync_copy(i1_hbm.at[b], i1_v)
    pltpu.sync_copy(w0_hbm.at[b], w0_v)
    pltpu.sync_copy(w1_hbm.at[b], w1_v)
    lanes = lax.iota(jnp.int32, 16)
    csz = QCH * PTS

    def _drain_out(c):
        pltpu.make_async_copy(
            out_v.at[pl.ds(0, csz)], dn_out.at[b, c, pl.ds(0, csz)], osem
        ).wait()

    for ci in range(DN_PER_W):
        c = j * DN_PER_W + ci
        pltpu.sync_copy(dense_hbm.at[b, c], tab_v)

        def _chunk(ch, _, c=c):
            par = (ch % 2) * csz

            @pl.when(ch >= 2)
            def _():
                _drain_out(c)

            @plsc.parallel_loop(0, NGRP)
            def _group(g):
                q0 = ch * QCH + g * GRP
                iv0 = i0_v[pl.ds(q0, GRP)] * PTS
                iv1 = i1_v[pl.ds(q0, GRP)] * PTS
                wv0 = w0_v[pl.ds(q0, GRP)]
                wv1 = w1_v[pl.ds(q0, GRP)]
                ob = par + g * (GRP * PTS) + lanes * PTS

                @plsc.parallel_loop(0, PTS, unroll=8)
                def _pt(p):
                    prot = jnp.bitwise_and(lanes + p, PTS - 1)
                    a0 = plsc.load_gather(tab_v, [iv0 + prot])
                    a1 = plsc.load_gather(tab_v, [iv1 + prot])
                    plsc.store_scatter(out_v, [ob + prot], wv0 * a0 + wv1 * a1)

            pltpu.async_copy(out_v.at[pl.ds(par, csz)],
                             dn_out.at[b, c, pl.ds(ch * csz, csz)], osem)
            return 0

        lax.fori_loop(0, NCHK, _chunk, 0)
        _drain_out(c)
        _drain_out(c)
    for ci in range(SP_PER_W):
        c = j * SP_PER_W + ci
        pltpu.sync_copy(sparse_hbm.at[b, c], srow_v)

        @plsc.parallel_loop(0, N_Q // GRP, unroll=4)
        def _sgroup(g):
            q0 = g * GRP
            a0 = plsc.load_gather(srow_v, [i0_v[pl.ds(q0, GRP)]])
            a1 = plsc.load_gather(srow_v, [i1_v[pl.ds(q0, GRP)]])
            sout_v[pl.ds(q0, GRP)] = (w0_v[pl.ds(q0, GRP)] * a0
                                      + w1_v[pl.ds(q0, GRP)] * a1)

        pltpu.sync_copy(sout_v, spg_out.at[b, c])


def _sc_gather(dense_flat, sparse_fea, i0, i1, w0, w1):
    fn = pl.kernel(
        _sc_gather_body,
        out_type=(jax.ShapeDtypeStruct((BS, DN_C, L_SEQ), jnp.float32),
                  jax.ShapeDtypeStruct((BS, SP_C, N_Q), jnp.float32)),
        mesh=plsc.VectorSubcoreMesh(core_axis_name="c", subcore_axis_name="s"),
        compiler_params=pltpu.CompilerParams(needs_layout_passes=False),
        scratch_types=[
            pltpu.VMEM((N_KEY * PTS,), jnp.float32),
            pltpu.VMEM((N_Q,), jnp.int32),
            pltpu.VMEM((N_Q,), jnp.int32),
            pltpu.VMEM((N_Q,), jnp.float32),
            pltpu.VMEM((N_Q,), jnp.float32),
            pltpu.VMEM((2 * QCH * PTS,), jnp.float32),
            pltpu.VMEM((N_KEY,), jnp.float32),
            pltpu.VMEM((N_Q,), jnp.float32),
            pltpu.SemaphoreType.DMA,
        ],
    )
    return fn(dense_flat, sparse_fea, i0, i1, w0, w1)


MB = 1024


def _mlp_body(w_ref, b_ref, x_ref, o_ref):
    x = x_ref[0]
    y = lax.dot_general(w_ref[...], x, (((1,), (0,)), ((), ())),
                        preferred_element_type=jnp.float32)
    y = (y + b_ref[...]) / jnp.sqrt(jnp.float32(1.0 + 1e-5))
    o_ref[0] = jax.nn.gelu(y)


def _mlp(sp_W, sp_b, spg):
    return pl.pallas_call(
        _mlp_body,
        grid=(BS, N_Q // MB),
        in_specs=[
            pl.BlockSpec((SP_C, SP_C), lambda b, m: (0, 0)),
            pl.BlockSpec((SP_C, 1), lambda b, m: (0, 0)),
            pl.BlockSpec((1, SP_C, MB), lambda b, m: (b, 0, m)),
        ],
        out_specs=pl.BlockSpec((1, SP_C, MB), lambda b, m: (b, 0, m)),
        out_shape=jax.ShapeDtypeStruct((BS, SP_C, N_Q), jnp.float32),
    )(sp_W, sp_b.reshape(SP_C, 1), spg)


CH = 8192
NCB = L_SEQ // CH


SEG = 128


def _conv_body(w_ref, b_ref, p_ref, x_ref, xm_ref, xp_ref, o_ref):
    kb = pl.program_id(1)
    x = x_ref[0]
    prev = xm_ref[0]
    nxt = xp_ref[0]
    cm = jnp.where(kb == 0, x[:, 0:1], prev[:, 127:128])
    cp = jnp.where(kb == NCB - 1, x[:, CH - 1:CH], nxt[:, 0:1])
    xm = jnp.concatenate([cm, x[:, :CH - 1]], axis=1)
    xp = jnp.concatenate([x[:, 1:], cp], axis=1)
    dot = lambda a, bb: lax.dot_general(a, bb, (((1,), (0,)), ((), ())),
                                        preferred_element_type=jnp.float32)
    w0 = w_ref[0:64]
    w1 = w_ref[64:128]
    w2 = w_ref[128:192]
    w3 = w_ref[192:256]
    ev = dot(w0, xm) + dot(w2, x)
    od = dot(w1, x) + dot(w3, xp)
    scale = 1.0 / jnp.sqrt(jnp.float32(1.0 + 1e-5))
    bias = b_ref[...]
    ev = jax.nn.gelu((ev + bias) * scale).astype(jnp.bfloat16)
    od = jax.nn.gelu((od + bias) * scale).astype(jnp.bfloat16)
    perm = p_ref[...]
    pieces = []
    for u in range(CH // SEG):
        eo = jnp.concatenate(
            [ev[:, u * SEG:(u + 1) * SEG], od[:, u * SEG:(u + 1) * SEG]], axis=1)
        pieces.append(lax.dot_general(eo, perm, (((1,), (0,)), ((), ())),
                                      preferred_element_type=jnp.float32))
    y = jnp.concatenate(pieces, axis=1)
    o_ref[0] = y.reshape(DN_C, 2 * CH // 64, 64)


def _conv(ct_W, ct_b, dn_cf):
    w_all = jnp.transpose(ct_W[:, :, ::-1], (2, 1, 0)).reshape(4 * DN_C, DN_C)
    tt = jnp.arange(2 * SEG)
    src = tt // 2 + SEG * (tt % 2)
    perm = (jnp.arange(2 * SEG)[:, None] == src[None, :]).astype(jnp.bfloat16)
    nhalo = CH // 128
    y = pl.pallas_call(
        _conv_body,
        grid=(BS, NCB),
        in_specs=[
            pl.BlockSpec((4 * DN_C, DN_C), lambda b, k: (0, 0)),
            pl.BlockSpec((DN_C, 1), lambda b, k: (0, 0)),
            pl.BlockSpec((2 * SEG, 2 * SEG), lambda b, k: (0, 0)),
            pl.BlockSpec((1, DN_C, CH), lambda b, k: (b, 0, k)),
            pl.BlockSpec((1, DN_C, 128),
                         lambda b, k: (b, 0, jnp.maximum(k * nhalo - 1, 0))),
            pl.BlockSpec((1, DN_C, 128),
                         lambda b, k: (b, 0, jnp.minimum((k + 1) * nhalo,
                                                         L_SEQ // 128 - 1))),
        ],
        out_specs=pl.BlockSpec((1, DN_C, 2 * CH // 64, 64), lambda b, k: (b, 0, k, 0)),
        out_shape=jax.ShapeDtypeStruct((BS, DN_C, N_Q, 2 * L_SEQ // N_Q), jnp.float32),
    )(w_all, ct_b.reshape(DN_C, 1), perm, dn_cf, dn_cf, dn_cf)
    return y


def kernel(sparse_fea, dense_fea, stk_coor, stk_coor_bef, sp_W, sp_b, ct_W, ct_b):
    i0, i1, w0, w1 = _knn(stk_coor_bef, stk_coor)
    dense_flat = dense_fea.reshape(BS, DN_C, N_KEY * PTS)
    dn_cf, spg = _sc_gather(dense_flat, sparse_fea, i0, i1, w0, w1)
    sp = _mlp(sp_W, sp_b, spg)
    y = _conv(ct_W, ct_b, dn_cf)
    return (sp, y)
